# baseline (speedup 1.0000x reference)
import functools

import jax
import jax.numpy as jnp
from jax import lax
from jax.experimental import pallas as pl
from jax.experimental.pallas import tpu as pltpu
from jax.experimental.pallas import tpu_sc as plsc

_D = 64
_CH = 128


@functools.lru_cache(maxsize=None)
def _build(batch: int):
    info = plsc.get_sparse_core_info()
    nw = info.num_cores * info.num_subcores
    bpw = batch // nw
    mesh = plsc.VectorSubcoreMesh(core_axis_name="c", subcore_axis_name="s")

    @functools.partial(
        pl.kernel,
        mesh=mesh,
        out_type=(
            jax.ShapeDtypeStruct((batch, _D), jnp.float32),
            jax.ShapeDtypeStruct((batch, _D), jnp.float32),
        ),
        scratch_types=[
            pltpu.VMEM((bpw,), jnp.int32),
            pltpu.VMEM((bpw,), jnp.int32),
            pltpu.VMEM((bpw, _D), jnp.float32),
            pltpu.VMEM((bpw, _D), jnp.float32),
            pltpu.SemaphoreType.DMA,
        ],
        compiler_params=pltpu.CompilerParams(use_tc_tiling_on_sc=False),
    )
    def od_gather(ori_hbm, dest_hbm, tbl_hbm, out_o_hbm, out_d_hbm,
                  idx_o, idx_d, rows_o, rows_d, sem):
        wid = lax.axis_index("s") * info.num_cores + lax.axis_index("c")
        base = wid * bpw
        pltpu.sync_copy(ori_hbm.at[pl.ds(base, bpw)], idx_o)
        pltpu.sync_copy(dest_hbm.at[pl.ds(base, bpw)], idx_d)

        for c in range(bpw // _CH):
            off = c * _CH
            pltpu.async_copy(
                tbl_hbm.at[idx_o.at[pl.ds(off, _CH)]],
                rows_o.at[pl.ds(off, _CH)], sem)
        for c in range(bpw // _CH):
            off = c * _CH
            pltpu.async_copy(
                tbl_hbm.at[idx_d.at[pl.ds(off, _CH)]],
                rows_d.at[pl.ds(off, _CH)], sem)

        pltpu.make_async_copy(
            out_o_hbm.at[pl.ds(0, bpw)], rows_o, sem).wait()
        pltpu.make_async_copy(
            out_o_hbm.at[pl.ds(0, bpw)], rows_d, sem).wait()

        pltpu.sync_copy(rows_o, out_o_hbm.at[pl.ds(base, bpw)])
        pltpu.sync_copy(rows_d, out_d_hbm.at[pl.ds(base, bpw)])

    return od_gather


def kernel(ori, dest, table):
    batch, = ori.shape
    return tuple(_build(batch)(ori, dest, table))
